# baseline (speedup 1.0000x reference)
import functools

import jax
import jax.numpy as jnp
from jax import lax
from jax.experimental import pallas as pl
from jax.experimental.pallas import tpu as pltpu
from jax.experimental.pallas import tpu_sc as plsc

_EMBED = 64
_WINDOW = 5
_HIDDEN = 128
_LABELS = 64
_BATCH = 16384

_NC = 2
_NS = 16
_NW = _NC * _NS
_ROWS = _BATCH * _WINDOW
_RPW = _ROWS // _NW
_CHUNK = 128
_NCHUNK = _RPW // _CHUNK


def _gather_body(idx_hbm, table_hbm, out_hbm, idx_v, rows_v, sem0, sem1):
    wid = lax.axis_index("s") * _NC + lax.axis_index("c")
    base = wid * _RPW
    pltpu.sync_copy(idx_hbm.at[wid], idx_v)
    sems = (sem0, sem1)
    cps = [None, None]
    cps[0] = pltpu.async_copy(table_hbm.at[idx_v.at[0]], rows_v.at[0], sems[0])
    for j in range(1, _NCHUNK):
        cps[j % 2] = pltpu.async_copy(
            table_hbm.at[idx_v.at[j]], rows_v.at[j % 2], sems[j % 2])
        k = j - 1
        cps[k % 2].wait()
        pltpu.sync_copy(rows_v.at[k % 2],
                        out_hbm.at[pl.ds(base + k * _CHUNK, _CHUNK)])
    k = _NCHUNK - 1
    cps[k % 2].wait()
    pltpu.sync_copy(rows_v.at[k % 2],
                    out_hbm.at[pl.ds(base + k * _CHUNK, _CHUNK)])


_sc_gather = functools.partial(
    pl.kernel,
    mesh=plsc.VectorSubcoreMesh(core_axis_name="c", subcore_axis_name="s"),
    out_type=jax.ShapeDtypeStruct((_ROWS, _EMBED), jnp.float32),
    scratch_types=[
        pltpu.VMEM((_NCHUNK, _CHUNK), jnp.int32),
        pltpu.VMEM((2, _CHUNK, _EMBED), jnp.float32),
        pltpu.SemaphoreType.DMA,
        pltpu.SemaphoreType.DMA,
    ],
    compiler_params=pltpu.CompilerParams(use_tc_tiling_on_sc=False),
)(_gather_body)


def _mlp_body(flat_ref, w1_ref, b1_ref, w2_ref, b2_ref, out_ref):
    h = jnp.tanh(
        jnp.dot(flat_ref[...], w1_ref[...],
                preferred_element_type=jnp.float32) + b1_ref[...])
    out_ref[...] = jnp.dot(
        h, w2_ref[...], preferred_element_type=jnp.float32) + b2_ref[...]


_BB = 1024


def _mlp(flat, W1, b1, W2, b2):
    return pl.pallas_call(
        _mlp_body,
        grid=(_BATCH // _BB,),
        in_specs=[
            pl.BlockSpec((_BB, _WINDOW * _EMBED), lambda i: (i, 0)),
            pl.BlockSpec((_WINDOW * _EMBED, _HIDDEN), lambda i: (0, 0)),
            pl.BlockSpec((1, _HIDDEN), lambda i: (0, 0)),
            pl.BlockSpec((_HIDDEN, _LABELS), lambda i: (0, 0)),
            pl.BlockSpec((1, _LABELS), lambda i: (0, 0)),
        ],
        out_specs=pl.BlockSpec((_BB, _LABELS), lambda i: (i, 0)),
        out_shape=jax.ShapeDtypeStruct((_BATCH, _LABELS), jnp.float32),
    )(flat, W1, b1, W2, b2)


def kernel(x, table, W1, b1, W2, b2):
    idx = x.reshape(_NW, _NCHUNK, _CHUNK)
    rows = _sc_gather(idx, table)
    flat = rows.reshape(_BATCH, _WINDOW * _EMBED)
    return _mlp(flat, W1, b1.reshape(1, _HIDDEN), W2, b2.reshape(1, _LABELS))
